# baseline (speedup 1.0000x reference)
import functools

import jax
import jax.numpy as jnp
from jax import lax
from jax.experimental import pallas as pl
from jax.experimental.pallas import tpu as pltpu
from jax.experimental.pallas import tpu_sc as plsc

_L = 16


@functools.lru_cache(maxsize=None)
def _make_sc_embed(B, S, D, interpret=False):
  try:
    info = plsc.get_sparse_core_info()
    NC, NS = info.num_cores, info.num_subcores
  except ValueError:
    NC, NS = 2, 16
  NW = NC * NS

  b_per_w = B // NW
  assert b_per_w * NW == B
  SEQ_PER_CHUNK = 2
  C = SEQ_PER_CHUNK * S
  n_chunks = b_per_w // C
  assert n_chunks * C == b_per_w
  assert D % _L == 0
  nvec = D // _L

  mesh = plsc.VectorSubcoreMesh(
      core_axis_name="c", subcore_axis_name="s",
      num_cores=NC, num_subcores=NS)

  @functools.partial(
      pl.kernel,
      mesh=mesh,
      out_type=jax.ShapeDtypeStruct((B, D), jnp.float32),
      scratch_types=[
          pltpu.VMEM((b_per_w,), jnp.int32),
          pltpu.VMEM((C, D), jnp.float32),
          pltpu.VMEM((C, D), jnp.float32),
          pltpu.SemaphoreType.DMA,
      ],
      compiler_params=pltpu.CompilerParams(use_tc_tiling_on_sc=False),
      interpret=interpret,
  )
  def sc_embed(word_hbm, idx_hbm, pos_hbm, out_hbm, idx_v, rows_v, pos_v, sem):
    wid = lax.axis_index("s") * NC + lax.axis_index("c")
    base = wid * b_per_w

    pltpu.sync_copy(idx_hbm.at[pl.ds(base, b_per_w)], idx_v)
    for r in range(SEQ_PER_CHUNK):
      pltpu.sync_copy(pos_hbm, pos_v.at[pl.ds(r * S, S)])

    @pl.loop(0, n_chunks)
    def _chunk(step):
      off = step * C
      pltpu.async_copy(
          word_hbm.at[idx_v.at[pl.ds(off, C)]], rows_v, sem).wait()
      @pl.loop(0, C)
      def _row(s):
        for k in range(nvec):
          sl = pl.ds(k * _L, _L)
          rows_v[s, sl] = rows_v[s, sl] + pos_v[s, sl]
      pltpu.sync_copy(rows_v, out_hbm.at[pl.ds(base + off, C)])

  return sc_embed


def kernel(x, word_table, pos_table):
  N, S = x.shape
  V, D = word_table.shape
  B = N * S
  flat_idx = x.reshape(B).astype(jnp.int32)
  sc = _make_sc_embed(B, S, D)
  out = sc(word_table, flat_idx, pos_table)
  return out.reshape(N, S, D)


# speedup vs baseline: 1.1145x; 1.1145x over previous
import functools

import jax
import jax.numpy as jnp
from jax import lax
from jax.experimental import pallas as pl
from jax.experimental.pallas import tpu as pltpu
from jax.experimental.pallas import tpu_sc as plsc

_L = 16


@functools.lru_cache(maxsize=None)
def _make_sc_embed(B, S, D, interpret=False):
  try:
    info = plsc.get_sparse_core_info()
    NC, NS = info.num_cores, info.num_subcores
  except ValueError:
    NC, NS = 2, 16
  NW = NC * NS

  b_per_w = B // NW
  assert b_per_w * NW == B
  C = S
  R = 4
  n_chunks = b_per_w // C
  assert n_chunks * C == b_per_w and n_chunks % R == 0
  assert D % _L == 0
  nvec = D // _L

  mesh = plsc.VectorSubcoreMesh(
      core_axis_name="c", subcore_axis_name="s",
      num_cores=NC, num_subcores=NS)

  @functools.partial(
      pl.kernel,
      mesh=mesh,
      out_type=jax.ShapeDtypeStruct((B, D), jnp.float32),
      scratch_types=[
          pltpu.VMEM((b_per_w,), jnp.int32),
          [pltpu.VMEM((C, D), jnp.float32) for _ in range(R)],
          pltpu.VMEM((C, D), jnp.float32),
          [pltpu.SemaphoreType.DMA for _ in range(R)],
          [pltpu.SemaphoreType.DMA for _ in range(R)],
      ],
      compiler_params=pltpu.CompilerParams(use_tc_tiling_on_sc=False),
      interpret=interpret,
  )
  def sc_embed(word_hbm, idx_hbm, pos_hbm, out_hbm,
               idx_v, rows, pos_v, gsem, osem):
    wid = lax.axis_index("s") * NC + lax.axis_index("c")
    base = wid * b_per_w

    pltpu.sync_copy(idx_hbm.at[pl.ds(base, b_per_w)], idx_v)
    pltpu.sync_copy(pos_hbm, pos_v)

    def gather(step, b):
      return pltpu.make_async_copy(
          word_hbm.at[idx_v.at[pl.ds(step * C, C)]], rows[b], gsem[b])

    def out_copy(step, b):
      return pltpu.make_async_copy(
          rows[b], out_hbm.at[pl.ds(base + step * C, C)], osem[b])

    gather(0, 0).start()
    gather(1, 1).start()

    @pl.loop(0, n_chunks, step=R)
    def _chunks(step0):
      for b in range(R):
        step = step0 + b
        gather(step, b).wait()
        @pl.loop(0, C)
        def _row(s):
          for k in range(nvec):
            sl = pl.ds(k * _L, _L)
            rows[b][s, sl] = rows[b][s, sl] + pos_v[s, sl]
        out_copy(step, b).start()
        nb = (b + 2) % R
        @pl.when(step + 2 < n_chunks)
        def _():
          @pl.when(step - 2 >= 0)
          def _():
            out_copy(step - 2, nb).wait()
          gather(step + 2, nb).start()
    for t in range(n_chunks - R, n_chunks):
      out_copy(t, t % R).wait()

  return sc_embed


def kernel(x, word_table, pos_table):
  N, S = x.shape
  V, D = word_table.shape
  B = N * S
  flat_idx = x.reshape(B).astype(jnp.int32)
  sc = _make_sc_embed(B, S, D)
  out = sc(word_table, flat_idx, pos_table)
  return out.reshape(N, S, D)
